# baseline (speedup 1.0000x reference)
import numpy as np
import jax
import jax.numpy as jnp
from jax.experimental import pallas as pl
from jax.experimental.pallas import tpu as pltpu
from jax.experimental.shard_map import shard_map
from jax.sharding import Mesh, PartitionSpec as P

_BB = 32


def _conv_body(x_ref, w1_ref, b1_ref, w2_ref, b2_ref, w3_ref, b3_ref,
               o_ref, xp_ref, p2_ref, p3_ref):
    f32 = jnp.float32
    bb = x_ref.shape[0]

    xp_ref[:, 0:1, :] = jnp.zeros((bb, 1, 72), f32)
    xp_ref[:, 65:66, :] = jnp.zeros((bb, 1, 72), f32)
    xp_ref[:, :, 0:4] = jnp.zeros((bb, 66, 4), f32)
    xp_ref[:, :, 68:72] = jnp.zeros((bb, 66, 4), f32)
    xp_ref[:, 1:65, 4:68] = x_ref[...].astype(f32)
    col1 = jnp.concatenate(
        [xp_ref[:, r:r + 64, :].reshape(bb * 64, 72) for r in range(3)],
        axis=1)
    t1 = jnp.dot(col1, w1_ref[...], preferred_element_type=f32) + b1_ref[...]
    h1 = jnp.maximum(t1[:, :512], t1[:, 512:])
    a1 = jnp.maximum(h1.reshape(bb, 32, 2, 512).max(axis=2), 0.0)

    p2_ref[:, 0:1, :] = jnp.zeros((bb, 1, 544), f32)
    p2_ref[:, 33:34, :] = jnp.zeros((bb, 1, 544), f32)
    p2_ref[:, :, 0:16] = jnp.zeros((bb, 34, 16), f32)
    p2_ref[:, :, 528:544] = jnp.zeros((bb, 34, 16), f32)
    p2_ref[:, 1:33, 16:528] = a1
    col2 = jnp.concatenate(
        [p2_ref[:, r:r + 32, :].reshape(bb * 32, 544) for r in range(3)],
        axis=1)
    t2 = jnp.dot(col2, w2_ref[...], preferred_element_type=f32) + b2_ref[...]
    h2 = jnp.maximum(t2[:, :512], t2[:, 512:])
    a2 = jnp.maximum(h2.reshape(bb, 16, 2, 512).max(axis=2), 0.0)

    p3_ref[:, 0:1, :] = jnp.zeros((bb, 1, 576), f32)
    p3_ref[:, 17:18, :] = jnp.zeros((bb, 1, 576), f32)
    p3_ref[:, :, 0:32] = jnp.zeros((bb, 18, 32), f32)
    p3_ref[:, :, 544:576] = jnp.zeros((bb, 18, 32), f32)
    p3_ref[:, 1:17, 32:544] = a2
    col3 = jnp.concatenate(
        [p3_ref[:, r:r + 16, :].reshape(bb * 16, 576) for r in range(3)],
        axis=1)
    t3 = jnp.dot(col3, w3_ref[...], preferred_element_type=f32) + b3_ref[...]
    h3 = jnp.maximum(t3[:, :512], t3[:, 512:])
    a3 = jnp.maximum(h3.reshape(bb, 8, 2, 512).max(axis=2), 0.0)

    o_ref[...] = a3


def _mlp_body(f_ref, w1_ref, b1_ref, w2_ref, b2_ref, o_ref):
    f32 = jnp.float32
    mb = f_ref.shape[0]
    acc = jnp.zeros((mb, 128), f32)
    for y in range(8):
        acc = acc + jnp.dot(f_ref[:, y, :], w1_ref[y],
                            preferred_element_type=f32)
    h = jnp.maximum(acc + b1_ref[...], 0.0)
    o_ref[...] = (jnp.dot(h, w2_ref[...], preferred_element_type=f32)
                  + b2_ref[...])


def _row_weight(conv_w, xpad, wout, off):
    f32 = jnp.float32
    co_n, ci_n = conv_w.shape[0], conv_w.shape[1]
    wt = jnp.transpose(conv_w, (2, 3, 1, 0)).astype(f32)
    eye = jnp.eye(wout, dtype=f32)
    shifts = jnp.stack(
        [jnp.pad(eye, ((dx + off, xpad - wout - dx - off), (0, 0)))
         for dx in range(3)])
    w = jnp.einsum('dpx,rdco->rpcxo', shifts, wt)
    w = w.reshape(3, xpad, ci_n, wout // 2, 2, co_n)
    w = jnp.transpose(w, (0, 1, 2, 4, 3, 5))
    return w.reshape(3 * xpad * ci_n, wout * co_n)


def _forward(x, w1, b1, w2, b2, w3, b3, wl1, bl1, wl2, bl2):
    f32 = jnp.float32
    Bl = x.shape[0]
    feats = pl.pallas_call(
        _conv_body,
        out_shape=jax.ShapeDtypeStruct((Bl, 8, 512), f32),
        grid=(Bl // _BB,),
        in_specs=[
            pl.BlockSpec((_BB, 64, 64), lambda b: (b, 0, 0)),
            pl.BlockSpec((216, 1024), lambda b: (0, 0)),
            pl.BlockSpec((1, 1024), lambda b: (0, 0)),
            pl.BlockSpec((1632, 1024), lambda b: (0, 0)),
            pl.BlockSpec((1, 1024), lambda b: (0, 0)),
            pl.BlockSpec((1728, 1024), lambda b: (0, 0)),
            pl.BlockSpec((1, 1024), lambda b: (0, 0)),
        ],
        out_specs=pl.BlockSpec((_BB, 8, 512), lambda b: (b, 0, 0)),
        scratch_shapes=[
            pltpu.VMEM((_BB, 66, 72), f32),
            pltpu.VMEM((_BB, 34, 544), f32),
            pltpu.VMEM((_BB, 18, 576), f32),
        ],
        compiler_params=pltpu.CompilerParams(
            dimension_semantics=("parallel",),
            vmem_limit_bytes=64 * 1024 * 1024),
    )(x, w1, b1, w2, b2, w3, b3)

    mb = min(256, Bl)
    return pl.pallas_call(
        _mlp_body,
        out_shape=jax.ShapeDtypeStruct((Bl, 128), f32),
        grid=(Bl // mb,),
        in_specs=[
            pl.BlockSpec((mb, 8, 512), lambda i: (i, 0, 0)),
            pl.BlockSpec((8, 512, 128), lambda i: (0, 0, 0)),
            pl.BlockSpec((1, 128), lambda i: (0, 0)),
            pl.BlockSpec((128, 128), lambda i: (0, 0)),
            pl.BlockSpec((1, 128), lambda i: (0, 0)),
        ],
        out_specs=pl.BlockSpec((mb, 128), lambda i: (i, 0)),
        compiler_params=pltpu.CompilerParams(
            dimension_semantics=("parallel",),
            vmem_limit_bytes=64 * 1024 * 1024),
    )(feats, wl1, bl1, wl2, bl2)


def kernel(x_nchw, conv1_w, conv1_b, conv2_w, conv2_b, conv3_w, conv3_b,
           linear1_w, linear1_b, linear2_w, linear2_b):
    B = x_nchw.shape[0]
    nc = linear2_w.shape[0]
    x = x_nchw.reshape(B, 64, 64).astype(jnp.bfloat16)

    w1 = _row_weight(conv1_w, 72, 64, 3)
    w2 = _row_weight(conv2_w, 34, 32, 0)
    w3 = _row_weight(conv3_w, 18, 16, 0)
    b1 = jnp.tile(conv1_b, 64).reshape(1, 1024)
    b2 = jnp.tile(conv2_b, 32).reshape(1, 1024)
    b3 = jnp.tile(conv3_b, 16).reshape(1, 1024)

    wl1 = jnp.transpose(linear1_w.reshape(128, 64, 8, 8),
                        (2, 3, 1, 0)).reshape(8, 512, 128)
    bl1 = linear1_b.reshape(1, 128)
    wl2 = jnp.pad(jnp.transpose(linear2_w), ((0, 0), (0, 128 - nc)))
    bl2 = jnp.pad(linear2_b, (0, 128 - nc)).reshape(1, 128)

    args = (w1, b1, w2, b2, w3, b3, wl1, bl1, wl2, bl2)
    devs = jax.devices()
    nd = len(devs)
    if nd > 1 and B % (nd * _BB) == 0:
        mesh = Mesh(np.array(devs), ("b",))
        fwd = shard_map(
            _forward, mesh=mesh,
            in_specs=(P("b"),) + (P(),) * len(args),
            out_specs=P("b"), check_rep=False)
        half = B // 2
        out = jnp.concatenate(
            [fwd(x[:half], *args), fwd(x[half:], *args)], axis=0)
    else:
        out = _forward(x, *args)
    return out[:, :nc]


# speedup vs baseline: 1.1478x; 1.1478x over previous
import numpy as np
import jax
import jax.numpy as jnp
from jax.experimental import pallas as pl
from jax.experimental.pallas import tpu as pltpu
from jax.experimental.shard_map import shard_map
from jax.sharding import Mesh, PartitionSpec as P

_BB = 32


def _conv_body(x_ref, w1_ref, b1_ref, w2_ref, b2_ref, w3_ref, b3_ref,
               o_ref, xp_ref, p2_ref, p3_ref):
    f32 = jnp.float32
    bb = x_ref.shape[0]

    xp_ref[:, 0:1, :] = jnp.zeros((bb, 1, 72), f32)
    xp_ref[:, 65:66, :] = jnp.zeros((bb, 1, 72), f32)
    xp_ref[:, :, 0:4] = jnp.zeros((bb, 66, 4), f32)
    xp_ref[:, :, 68:72] = jnp.zeros((bb, 66, 4), f32)
    xp_ref[:, 1:65, 4:68] = x_ref[...].astype(f32)
    col1 = jnp.concatenate(
        [xp_ref[:, r:r + 64, :].reshape(bb * 64, 72) for r in range(3)],
        axis=1)
    t1 = jnp.dot(col1, w1_ref[...], preferred_element_type=f32) + b1_ref[...]
    h1 = jnp.maximum(t1[:, :512], t1[:, 512:])
    a1 = jnp.maximum(h1.reshape(bb, 32, 2, 512).max(axis=2), 0.0)

    p2_ref[:, 0:1, :] = jnp.zeros((bb, 1, 544), f32)
    p2_ref[:, 33:34, :] = jnp.zeros((bb, 1, 544), f32)
    p2_ref[:, :, 0:16] = jnp.zeros((bb, 34, 16), f32)
    p2_ref[:, :, 528:544] = jnp.zeros((bb, 34, 16), f32)
    p2_ref[:, 1:33, 16:528] = a1
    col2 = jnp.concatenate(
        [p2_ref[:, r:r + 32, :].reshape(bb * 32, 544) for r in range(3)],
        axis=1)
    t2 = jnp.dot(col2, w2_ref[...], preferred_element_type=f32) + b2_ref[...]
    h2 = jnp.maximum(t2[:, :512], t2[:, 512:])
    a2 = jnp.maximum(h2.reshape(bb, 16, 2, 512).max(axis=2), 0.0)

    p3_ref[:, 0:1, :] = jnp.zeros((bb, 1, 576), f32)
    p3_ref[:, 17:18, :] = jnp.zeros((bb, 1, 576), f32)
    p3_ref[:, :, 0:32] = jnp.zeros((bb, 18, 32), f32)
    p3_ref[:, :, 544:576] = jnp.zeros((bb, 18, 32), f32)
    p3_ref[:, 1:17, 32:544] = a2
    col3 = jnp.concatenate(
        [p3_ref[:, r:r + 16, :].reshape(bb * 16, 576) for r in range(3)],
        axis=1)
    t3 = jnp.dot(col3, w3_ref[...], preferred_element_type=f32) + b3_ref[...]
    h3 = jnp.maximum(t3[:, :512], t3[:, 512:])
    a3 = jnp.maximum(h3.reshape(bb, 8, 2, 512).max(axis=2), 0.0)

    o_ref[...] = a3


def _mlp_body(f_ref, w1_ref, b1_ref, w2_ref, b2_ref, o_ref):
    f32 = jnp.float32
    mb = f_ref.shape[0]
    acc = jnp.zeros((mb, 128), f32)
    for y in range(8):
        acc = acc + jnp.dot(f_ref[:, y, :], w1_ref[y],
                            preferred_element_type=f32)
    h = jnp.maximum(acc + b1_ref[...], 0.0)
    o_ref[...] = (jnp.dot(h, w2_ref[...], preferred_element_type=f32)
                  + b2_ref[...])


def _row_weight(conv_w, xpad, wout, off):
    f32 = jnp.float32
    co_n, ci_n = conv_w.shape[0], conv_w.shape[1]
    wt = jnp.transpose(conv_w, (2, 3, 1, 0)).astype(f32)
    eye = jnp.eye(wout, dtype=f32)
    shifts = jnp.stack(
        [jnp.pad(eye, ((dx + off, xpad - wout - dx - off), (0, 0)))
         for dx in range(3)])
    w = jnp.einsum('dpx,rdco->rpcxo', shifts, wt)
    w = w.reshape(3, xpad, ci_n, wout // 2, 2, co_n)
    w = jnp.transpose(w, (0, 1, 2, 4, 3, 5))
    return w.reshape(3 * xpad * ci_n, wout * co_n)


def _forward(x, w1, b1, w2, b2, w3, b3, wl1, bl1, wl2, bl2):
    f32 = jnp.float32
    Bl = x.shape[0]
    feats = pl.pallas_call(
        _conv_body,
        out_shape=jax.ShapeDtypeStruct((Bl, 8, 512), f32),
        grid=(Bl // _BB,),
        in_specs=[
            pl.BlockSpec((_BB, 64, 64), lambda b: (b, 0, 0)),
            pl.BlockSpec((216, 1024), lambda b: (0, 0)),
            pl.BlockSpec((1, 1024), lambda b: (0, 0)),
            pl.BlockSpec((1632, 1024), lambda b: (0, 0)),
            pl.BlockSpec((1, 1024), lambda b: (0, 0)),
            pl.BlockSpec((1728, 1024), lambda b: (0, 0)),
            pl.BlockSpec((1, 1024), lambda b: (0, 0)),
        ],
        out_specs=pl.BlockSpec((_BB, 8, 512), lambda b: (b, 0, 0)),
        scratch_shapes=[
            pltpu.VMEM((_BB, 66, 72), f32),
            pltpu.VMEM((_BB, 34, 544), f32),
            pltpu.VMEM((_BB, 18, 576), f32),
        ],
        compiler_params=pltpu.CompilerParams(
            dimension_semantics=("parallel",),
            vmem_limit_bytes=64 * 1024 * 1024),
    )(x, w1, b1, w2, b2, w3, b3)

    mb = min(256, Bl)
    return pl.pallas_call(
        _mlp_body,
        out_shape=jax.ShapeDtypeStruct((Bl, 128), f32),
        grid=(Bl // mb,),
        in_specs=[
            pl.BlockSpec((mb, 8, 512), lambda i: (i, 0, 0)),
            pl.BlockSpec((8, 512, 128), lambda i: (0, 0, 0)),
            pl.BlockSpec((1, 128), lambda i: (0, 0)),
            pl.BlockSpec((128, 128), lambda i: (0, 0)),
            pl.BlockSpec((1, 128), lambda i: (0, 0)),
        ],
        out_specs=pl.BlockSpec((mb, 128), lambda i: (i, 0)),
        compiler_params=pltpu.CompilerParams(
            dimension_semantics=("parallel",),
            vmem_limit_bytes=64 * 1024 * 1024),
    )(feats, wl1, bl1, wl2, bl2)


def kernel(x_nchw, conv1_w, conv1_b, conv2_w, conv2_b, conv3_w, conv3_b,
           linear1_w, linear1_b, linear2_w, linear2_b):
    B = x_nchw.shape[0]
    nc = linear2_w.shape[0]
    x = x_nchw.reshape(B, 64, 64).astype(jnp.bfloat16)

    w1 = _row_weight(conv1_w, 72, 64, 3)
    w2 = _row_weight(conv2_w, 34, 32, 0)
    w3 = _row_weight(conv3_w, 18, 16, 0)
    b1 = jnp.tile(conv1_b, 64).reshape(1, 1024)
    b2 = jnp.tile(conv2_b, 32).reshape(1, 1024)
    b3 = jnp.tile(conv3_b, 16).reshape(1, 1024)

    wl1 = jnp.transpose(linear1_w.reshape(128, 64, 8, 8),
                        (2, 3, 1, 0)).reshape(8, 512, 128)
    bl1 = linear1_b.reshape(1, 128)
    wl2 = jnp.pad(jnp.transpose(linear2_w), ((0, 0), (0, 128 - nc)))
    bl2 = jnp.pad(linear2_b, (0, 128 - nc)).reshape(1, 128)

    args = (w1, b1, w2, b2, w3, b3, wl1, bl1, wl2, bl2)
    devs = jax.devices()
    nd = len(devs)
    if nd > 1 and B % (nd * _BB) == 0:
        mesh = Mesh(np.array(devs), ("b",))
        fwd = shard_map(
            _forward, mesh=mesh,
            in_specs=(P("b"),) + (P(),) * len(args),
            out_specs=P("b"), check_rep=False)
        out = fwd(x, *args)
    else:
        out = _forward(x, *args)
    return out[:, :nc]
